# baseline (speedup 1.0000x reference)
import functools

import jax
import jax.numpy as jnp
from jax import lax
from jax.experimental import pallas as pl
from jax.experimental.pallas import tpu as pltpu, tpu_sc as plsc

G, N, E = 4, 2500, 40000
H = 128
NP = 2560
EPT = E // 16
EPTP = NP
IC = 64
NCH = 2 * EPTP // IC
ROWS_PER_TILE = 2 * NP // 16

_mesh = plsc.VectorSubcoreMesh(core_axis_name="c", subcore_axis_name="s")


def _deg_body(srcL_hbm, dstL_hbm, dego_hbm, degi_hbm,
              sidx, didx, ones_v, zb, stage, sco, sci, osems, isems):
    c = lax.axis_index("c")
    t = lax.axis_index("s")

    for j in range(8):
        ones_v[pl.ds(j * 16, 16)] = jnp.ones((16,), jnp.float32)
    def zb16(i, _):
        zb[pl.ds(i * 16, 16)] = jnp.zeros((16,), jnp.float32)
        return 0
    lax.fori_loop(0, ROWS_PER_TILE // 16, zb16, 0)
    pltpu.sync_copy(zb, sco.at[pl.ds(t * ROWS_PER_TILE, ROWS_PER_TILE)])
    pltpu.sync_copy(zb, sci.at[pl.ds(t * ROWS_PER_TILE, ROWS_PER_TILE)])

    pltpu.sync_copy(srcL_hbm.at[c, t], sidx)
    pltpu.sync_copy(dstL_hbm.at[c, t], didx)
    plsc.subcore_barrier()

    ones16 = ones_v.at[pl.ds(0, IC)]

    def count(i, _):
        @pl.when(i >= 4)
        def _():
            pltpu.make_async_copy(ones16, sco.at[sidx.at[i - 4]],
                                  osems.at[(i - 4) % 4]).wait()
            pltpu.make_async_copy(ones16, sci.at[didx.at[i - 4]],
                                  isems.at[(i - 4) % 4]).wait()
        pltpu.async_copy(ones16, sco.at[sidx.at[i]], osems.at[i % 4],
                         add=True)
        pltpu.async_copy(ones16, sci.at[didx.at[i]], isems.at[i % 4],
                         add=True)
        return 0
    lax.fori_loop(0, NCH, count, 0)

    def drain(i, _):
        pltpu.make_async_copy(ones16, sco.at[sidx.at[i]],
                              osems.at[i % 4]).wait()
        pltpu.make_async_copy(ones16, sci.at[didx.at[i]],
                              isems.at[i % 4]).wait()
        return 0
    lax.fori_loop(NCH - 4, NCH, drain, 0)
    plsc.subcore_barrier()

    pltpu.sync_copy(sco.at[pl.ds(t * ROWS_PER_TILE, ROWS_PER_TILE)], stage)
    pltpu.sync_copy(stage, dego_hbm.at[c, t])
    pltpu.sync_copy(sci.at[pl.ds(t * ROWS_PER_TILE, ROWS_PER_TILE)], stage)
    pltpu.sync_copy(stage, degi_hbm.at[c, t])


_deg_call = pl.kernel(
    _deg_body,
    out_type=[jax.ShapeDtypeStruct((2, 16, ROWS_PER_TILE), jnp.float32),
              jax.ShapeDtypeStruct((2, 16, ROWS_PER_TILE), jnp.float32)],
    mesh=_mesh,
    scratch_types=[
        pltpu.VMEM((NCH, IC), jnp.int32),
        pltpu.VMEM((NCH, IC), jnp.int32),
        pltpu.VMEM((128,), jnp.float32),
        pltpu.VMEM((ROWS_PER_TILE,), jnp.float32),
        pltpu.VMEM((ROWS_PER_TILE,), jnp.float32),
        pltpu.VMEM_SHARED((2 * NP,), jnp.float32),
        pltpu.VMEM_SHARED((2 * NP,), jnp.float32),
        pltpu.SemaphoreType.DMA((4,)),
        pltpu.SemaphoreType.DMA((4,)),
    ],
)


def _msg_body(table_hbm, srcC_hbm, dstL_hbm, out_hbm,
              sidx, didx, bufs, zb, acc, gsems, ssems):
    c = lax.axis_index("c")
    t = lax.axis_index("s")

    pltpu.async_copy(srcC_hbm.at[c, t], sidx, gsems.at[0])
    pltpu.async_copy(dstL_hbm.at[c, t], didx, gsems.at[1])

    def zfill(i, _):
        zb[i // 8, pl.ds((i % 8) * 16, 16)] = jnp.zeros((16,), jnp.float32)
        return 0
    lax.fori_loop(0, 32 * 8, zfill, 0)
    for f in range(10):
        pltpu.sync_copy(zb, acc.at[pl.ds(t * ROWS_PER_TILE + f * 32, 32)])
    pltpu.make_async_copy(srcC_hbm.at[c, t], sidx, gsems.at[0]).wait()
    pltpu.make_async_copy(dstL_hbm.at[c, t], didx, gsems.at[1]).wait()
    plsc.subcore_barrier()

    def gath(i, b):
        return pltpu.async_copy(table_hbm.at[sidx.at[i]], bufs.at[b],
                                gsems.at[b])

    def scat(i, b):
        return pltpu.async_copy(bufs.at[b], acc.at[didx.at[i]],
                                ssems.at[b], add=True)

    for i in range(7):
        gath(i, i)

    def group8(j, _):
        for b in range(8):
            i = j * 8 + b
            pltpu.make_async_copy(table_hbm.at[sidx.at[i]], bufs.at[b],
                                  gsems.at[b]).wait()
            nb = (b + 7) % 8
            @pl.when(i > 0)
            def _():
                pltpu.make_async_copy(bufs.at[nb], acc.at[didx.at[i - 1]],
                                      ssems.at[nb]).wait()
            @pl.when(i + 7 < NCH)
            def _():
                gath(i + 7, nb)
            scat(i, b)
        return 0
    lax.fori_loop(0, NCH // 8, group8, 0)
    pltpu.make_async_copy(bufs.at[7], acc.at[didx.at[NCH - 1]],
                          ssems.at[7]).wait()
    plsc.subcore_barrier()

    for f in range(5):
        base = t * ROWS_PER_TILE + f * 64
        pltpu.sync_copy(acc.at[pl.ds(base, 64)], bufs.at[f])
        pltpu.async_copy(bufs.at[f], out_hbm.at[c, pl.ds(base, 64)],
                         gsems.at[f])
    for f in range(5):
        pltpu.make_async_copy(bufs.at[f],
                              out_hbm.at[c, pl.ds(t * ROWS_PER_TILE + f * 64, 64)],
                              gsems.at[f]).wait()


_msg_call = pl.kernel(
    _msg_body,
    out_type=jax.ShapeDtypeStruct((2, 2 * NP, H), jnp.float32),
    mesh=_mesh,
    scratch_types=[
        pltpu.VMEM((NCH, IC), jnp.int32),
        pltpu.VMEM((NCH, IC), jnp.int32),
        pltpu.VMEM((8, IC, H), jnp.float32),
        pltpu.VMEM((32, H), jnp.float32),
        pltpu.VMEM_SHARED((2 * NP, H), jnp.float32),
        pltpu.SemaphoreType.DMA((8,)),
        pltpu.SemaphoreType.DMA((8,)),
    ],
)


GNP = G * NP


_LRB = 1280
_blk_spec = pl.BlockSpec((1, _LRB, H), lambda g, r: (g, r, 0))
_scale_spec = pl.BlockSpec((1, _LRB, 8), lambda g, r: (g, r, 0))
_W_spec = pl.BlockSpec((H, H), lambda g, r: (0, 0))
_bias_spec = pl.BlockSpec((1, H), lambda g, r: (0, 0))


def _prep_body(xpad_ref, cnto_ref, cnti_ref, t0_ref, ro_ref, ri_ref):
    ro = lax.rsqrt(jnp.maximum(cnto_ref[...], 1.0))
    ro_ref[...] = ro
    ri_ref[...] = lax.rsqrt(jnp.maximum(cnti_ref[...], 1.0))
    t0_ref[...] = xpad_ref[...] * ro[:, :, 0:1]


def _prep_call(xpad, cnto3, cnti3):
    return pl.pallas_call(
        _prep_body,
        grid=(G, NP // _LRB),
        in_specs=[_blk_spec, _scale_spec, _scale_spec],
        out_specs=[_blk_spec, _scale_spec, _scale_spec],
        out_shape=[jax.ShapeDtypeStruct((G, NP, H), jnp.float32),
                   jax.ShapeDtypeStruct((G, NP, 8), jnp.float32),
                   jax.ShapeDtypeStruct((G, NP, 8), jnp.float32)],
    )(xpad, cnto3, cnti3)


def _layer_body(agg_ref, ri_ref, ro_ref, W_ref, b_ref, out_ref, *, do_relu):
    z = (agg_ref[0] * ri_ref[0, :, 0:1]) @ W_ref[...] + b_ref[...]
    if do_relu:
        z = jnp.maximum(z, 0.0)
    out_ref[0] = z * ro_ref[0, :, 0:1]


def _layer_call(agg, ri3, ro3, W, b, do_relu):
    return pl.pallas_call(
        functools.partial(_layer_body, do_relu=do_relu),
        grid=(G, NP // _LRB),
        in_specs=[_blk_spec, _scale_spec, _scale_spec, _W_spec, _bias_spec],
        out_specs=_blk_spec,
        out_shape=jax.ShapeDtypeStruct((G, NP, H), jnp.float32),
    )(agg, ri3, ro3, W, b)


def _last_body(agg_ref, ri_ref, W_ref, b_ref, Wq_ref, bq_ref, Wk_ref, bk_ref,
               h_ref, q_ref, k_ref):
    z = jnp.maximum((agg_ref[0] * ri_ref[0, :, 0:1]) @ W_ref[...]
                    + b_ref[...], 0.0)
    h_ref[0] = z
    q_ref[0] = z @ Wq_ref[...] + bq_ref[...]
    k_ref[0] = z @ Wk_ref[...] + bk_ref[...]


def _last_call(agg, ri3, W, b, Wq, bq, Wk, bk):
    return pl.pallas_call(
        _last_body,
        grid=(G, NP // _LRB),
        in_specs=[_blk_spec, _scale_spec, _W_spec, _bias_spec,
                  _W_spec, _bias_spec, _W_spec, _bias_spec],
        out_specs=[_blk_spec] * 3,
        out_shape=[jax.ShapeDtypeStruct((G, NP, H), jnp.float32)] * 3,
    )(agg, ri3, W, b, Wq, bq, Wk, bk)


_BR = 256
_NB = NP // _BR


def _att_body(q_ref, k_ref, hfull_ref, hblk_ref, Wv_ref, bv_ref, out_ref):
    br = pl.program_id(1)
    v = hfull_ref[0] @ Wv_ref[...] + bv_ref[0, 0]
    s = q_ref[0] @ k_ref[0].T
    jmask = lax.broadcasted_iota(jnp.int32, (1, NP), 1) < N
    s = jnp.where(jmask, s, -1e30)
    m = jnp.max(s, axis=1, keepdims=True)
    p = jnp.exp(s - m)
    denom = jnp.sum(p, axis=1, keepdims=True)
    fw = (p @ v) / denom
    imask = (lax.broadcasted_iota(jnp.int32, (_BR, 1), 0) + br * _BR) < N
    fw = jnp.where(imask, fw, 0.0)
    contrib = jnp.sum(hblk_ref[0] * fw, axis=0)

    @pl.when(br == 0)
    def _():
        out_ref[...] = jnp.zeros((1, 8, H), jnp.float32)
    out_ref[0, 0, :] += contrib


def _att_call(q, k, h, Wv, bv):
    return pl.pallas_call(
        _att_body,
        grid=(G, _NB),
        in_specs=[
            pl.BlockSpec((1, _BR, H), lambda g, b: (g, b, 0)),
            pl.BlockSpec((1, NP, H), lambda g, b: (g, 0, 0)),
            pl.BlockSpec((1, NP, H), lambda g, b: (g, 0, 0)),
            pl.BlockSpec((1, _BR, H), lambda g, b: (g, b, 0)),
            pl.BlockSpec((H, 1), lambda g, b: (0, 0)),
            pl.BlockSpec((1, 1), lambda g, b: (0, 0)),
        ],
        out_specs=pl.BlockSpec((1, 8, H), lambda g, b: (g, 0, 0)),
        out_shape=jax.ShapeDtypeStruct((G, 8, H), jnp.float32),
    )(q, k, h, h, Wv, bv)[:, 0, :]


def _mlp_body(z_ref, W1_ref, b1_ref, g1_ref, beta1_ref, W2_ref, b2_ref,
              g2_ref, beta2_ref, W3_ref, b3_ref, out_ref):
    def ln_leaky(z, g, b):
        mu = jnp.mean(z, axis=-1, keepdims=True)
        var = jnp.mean((z - mu) ** 2, axis=-1, keepdims=True)
        z = (z - mu) * lax.rsqrt(var + 1e-5) * g + b
        return jnp.where(z > 0, z, 0.01 * z)

    z1 = ln_leaky(z_ref[...] @ W1_ref[...] + b1_ref[...], g1_ref[...], beta1_ref[...])
    z2 = ln_leaky(z1 @ W2_ref[...] + b2_ref[...], g2_ref[...], beta2_ref[...])
    out_ref[...] = z2 @ W3_ref[...] + b3_ref[...]


def _mlp_call(z, W1, b1, g1, beta1, W2, b2, g2, beta2, W3, b3):
    return pl.pallas_call(
        _mlp_body,
        out_shape=jax.ShapeDtypeStruct((G, 1), jnp.float32),
    )(z, W1, b1, g1, beta1, W2, b2, g2, beta2, W3, b3)


def kernel(x, edge_index, add_features, W0, b0, W_gcr, b_gcr, Wq, bq, Wk, bk,
           Wv, bv, W1, b1, g1, beta1, W2, b2, g2, beta2, W3, b3):
    src = edge_index[:, 0, :].astype(jnp.int32).reshape(G, 16, EPT)
    dst = edge_index[:, 1, :].astype(jnp.int32).reshape(G, 16, EPT)

    padv = (N + (jnp.arange(EPTP - EPT, dtype=jnp.int32) % 16))
    padv = jnp.broadcast_to(padv, (G, 16, EPTP - EPT))
    srcp = jnp.concatenate([src, padv], axis=2)
    dstp = jnp.concatenate([dst, padv], axis=2)

    gidx = jnp.arange(G, dtype=jnp.int32)[:, None, None]
    slot = (gidx % 2)
    srcC = srcp + gidx * NP
    srcL = srcp + slot * NP
    dstL = dstp + slot * NP

    def regroup(a):
        return (a.reshape(2, 2, 16, EPTP).transpose(0, 2, 1, 3)
                 .reshape(2, 16, NCH, IC))
    srcC4, srcL4, dstL4 = regroup(srcC), regroup(srcL), regroup(dstL)

    dego2, degi2 = _deg_call(srcL4, dstL4)
    cnto3 = jnp.broadcast_to(dego2.reshape(G, NP, 1), (G, NP, 8))
    cnti3 = jnp.broadcast_to(degi2.reshape(G, NP, 1), (G, NP, 8))

    xpad = jnp.pad(x, ((0, 0), (0, NP - N), (0, 0)))
    table0, ro, ri = _prep_call(xpad, cnto3, cnti3)

    Ws = [W0] + [W_gcr[i] for i in range(10)]
    bs = [jnp.broadcast_to(b0, (1, H))] + \
         [jnp.broadcast_to(b_gcr[i], (1, H)) for i in range(10)]

    table = table0.reshape(G * NP, H)
    for l in range(10):
        agg = _msg_call(table, srcC4, dstL4).reshape(G, NP, H)
        table = _layer_call(agg, ri, ro, Ws[l], bs[l],
                            do_relu=(l != 0)).reshape(G * NP, H)
    agg = _msg_call(table, srcC4, dstL4).reshape(G, NP, H)
    h, q, k = _last_call(agg, ri, Ws[10], bs[10],
                         Wq, jnp.broadcast_to(bq, (1, H)),
                         Wk, jnp.broadcast_to(bk, (1, H)))

    feats = _att_call(q, k, h, Wv, jnp.reshape(bv, (1, 1)))
    z = jnp.concatenate([feats, add_features], axis=1)
    out2 = _mlp_call(z, W1, jnp.broadcast_to(b1, (1, 1024)), g1, beta1,
                     W2, jnp.broadcast_to(b2, (1, 512)), g2, beta2,
                     W3, jnp.reshape(b3, (1, 1)))
    return out2[:, 0]


# speedup vs baseline: 1.0581x; 1.0581x over previous
import functools

import jax
import jax.numpy as jnp
from jax import lax
from jax.experimental import pallas as pl
from jax.experimental.pallas import tpu as pltpu, tpu_sc as plsc

G, N, E = 4, 2500, 40000
H = 128
NP = 2560
EPT = E // 16
EPTP = NP
IC = 64
NCH = 2 * EPTP // IC
ROWS_PER_TILE = 2 * NP // 16

_mesh = plsc.VectorSubcoreMesh(core_axis_name="c", subcore_axis_name="s")


def _deg_body(srcL_hbm, dstL_hbm, dego_hbm, degi_hbm,
              sidx, didx, ones_v, zb, stage, sco, sci, osems, isems):
    c = lax.axis_index("c")
    t = lax.axis_index("s")

    for j in range(8):
        ones_v[pl.ds(j * 16, 16)] = jnp.ones((16,), jnp.float32)
    def zb16(i, _):
        zb[pl.ds(i * 16, 16)] = jnp.zeros((16,), jnp.float32)
        return 0
    lax.fori_loop(0, ROWS_PER_TILE // 16, zb16, 0)
    pltpu.sync_copy(zb, sco.at[pl.ds(t * ROWS_PER_TILE, ROWS_PER_TILE)])
    pltpu.sync_copy(zb, sci.at[pl.ds(t * ROWS_PER_TILE, ROWS_PER_TILE)])

    pltpu.sync_copy(srcL_hbm.at[c, t], sidx)
    pltpu.sync_copy(dstL_hbm.at[c, t], didx)
    plsc.subcore_barrier()

    ones16 = ones_v.at[pl.ds(0, IC)]

    def count(i, _):
        @pl.when(i >= 4)
        def _():
            pltpu.make_async_copy(ones16, sco.at[sidx.at[i - 4]],
                                  osems.at[(i - 4) % 4]).wait()
            pltpu.make_async_copy(ones16, sci.at[didx.at[i - 4]],
                                  isems.at[(i - 4) % 4]).wait()
        pltpu.async_copy(ones16, sco.at[sidx.at[i]], osems.at[i % 4],
                         add=True)
        pltpu.async_copy(ones16, sci.at[didx.at[i]], isems.at[i % 4],
                         add=True)
        return 0
    lax.fori_loop(0, NCH, count, 0)

    def drain(i, _):
        pltpu.make_async_copy(ones16, sco.at[sidx.at[i]],
                              osems.at[i % 4]).wait()
        pltpu.make_async_copy(ones16, sci.at[didx.at[i]],
                              isems.at[i % 4]).wait()
        return 0
    lax.fori_loop(NCH - 4, NCH, drain, 0)
    plsc.subcore_barrier()

    pltpu.sync_copy(sco.at[pl.ds(t * ROWS_PER_TILE, ROWS_PER_TILE)], stage)
    pltpu.sync_copy(stage, dego_hbm.at[c, t])
    pltpu.sync_copy(sci.at[pl.ds(t * ROWS_PER_TILE, ROWS_PER_TILE)], stage)
    pltpu.sync_copy(stage, degi_hbm.at[c, t])


_deg_call = pl.kernel(
    _deg_body,
    out_type=[jax.ShapeDtypeStruct((2, 16, ROWS_PER_TILE), jnp.float32),
              jax.ShapeDtypeStruct((2, 16, ROWS_PER_TILE), jnp.float32)],
    mesh=_mesh,
    scratch_types=[
        pltpu.VMEM((NCH, IC), jnp.int32),
        pltpu.VMEM((NCH, IC), jnp.int32),
        pltpu.VMEM((128,), jnp.float32),
        pltpu.VMEM((ROWS_PER_TILE,), jnp.float32),
        pltpu.VMEM((ROWS_PER_TILE,), jnp.float32),
        pltpu.VMEM_SHARED((2 * NP,), jnp.float32),
        pltpu.VMEM_SHARED((2 * NP,), jnp.float32),
        pltpu.SemaphoreType.DMA((4,)),
        pltpu.SemaphoreType.DMA((4,)),
    ],
)


def _msg_body(table_hbm, srcC_hbm, dstL_hbm, out_hbm,
              sidx, didx, bufs, zb, acc, gsems, ssems):
    c = lax.axis_index("c")
    t = lax.axis_index("s")

    pltpu.async_copy(srcC_hbm.at[c, t], sidx, gsems.at[0])
    pltpu.async_copy(dstL_hbm.at[c, t], didx, gsems.at[1])

    def zfill(i, _):
        zb[i // 8, pl.ds((i % 8) * 16, 16)] = jnp.zeros((16,), jnp.float32)
        return 0
    lax.fori_loop(0, 32 * 8, zfill, 0)
    for f in range(10):
        pltpu.sync_copy(zb, acc.at[pl.ds(t * ROWS_PER_TILE + f * 32, 32)])
    pltpu.make_async_copy(srcC_hbm.at[c, t], sidx, gsems.at[0]).wait()
    pltpu.make_async_copy(dstL_hbm.at[c, t], didx, gsems.at[1]).wait()
    plsc.subcore_barrier()

    def gath(i, b):
        return pltpu.async_copy(table_hbm.at[sidx.at[i]], bufs.at[b],
                                gsems.at[b])

    def scat(i, b):
        return pltpu.async_copy(bufs.at[b], acc.at[didx.at[i]],
                                ssems.at[b], add=True)

    for i in range(7):
        gath(i, i)

    def group8(j, _):
        for b in range(8):
            i = j * 8 + b
            pltpu.make_async_copy(table_hbm.at[sidx.at[i]], bufs.at[b],
                                  gsems.at[b]).wait()
            nb = (b + 7) % 8
            @pl.when(i > 0)
            def _():
                pltpu.make_async_copy(bufs.at[nb], acc.at[didx.at[i - 1]],
                                      ssems.at[nb]).wait()
            @pl.when(i + 7 < NCH)
            def _():
                gath(i + 7, nb)
            scat(i, b)
        return 0
    lax.fori_loop(0, NCH // 8, group8, 0)
    pltpu.make_async_copy(bufs.at[7], acc.at[didx.at[NCH - 1]],
                          ssems.at[7]).wait()
    plsc.subcore_barrier()

    for f in range(5):
        base = t * ROWS_PER_TILE + f * 64
        pltpu.sync_copy(acc.at[pl.ds(base, 64)], bufs.at[f])
        pltpu.async_copy(bufs.at[f], out_hbm.at[c, pl.ds(base, 64)],
                         gsems.at[f])
    for f in range(5):
        pltpu.make_async_copy(bufs.at[f],
                              out_hbm.at[c, pl.ds(t * ROWS_PER_TILE + f * 64, 64)],
                              gsems.at[f]).wait()


_msg_call = pl.kernel(
    _msg_body,
    out_type=jax.ShapeDtypeStruct((2, 2 * NP, H), jnp.float32),
    mesh=_mesh,
    scratch_types=[
        pltpu.VMEM((NCH, IC), jnp.int32),
        pltpu.VMEM((NCH, IC), jnp.int32),
        pltpu.VMEM((8, IC, H), jnp.float32),
        pltpu.VMEM((32, H), jnp.float32),
        pltpu.VMEM_SHARED((2 * NP, H), jnp.float32),
        pltpu.SemaphoreType.DMA((8,)),
        pltpu.SemaphoreType.DMA((8,)),
    ],
)


GNP = G * NP


def _prep_body(xpad_ref, cnto_ref, cnti_ref, t0_ref, ro_ref, ri_ref):
    ro = lax.rsqrt(jnp.maximum(cnto_ref[...], 1.0))
    ri = lax.rsqrt(jnp.maximum(cnti_ref[...], 1.0))
    ro_ref[...] = ro
    ri_ref[...] = ri
    t0_ref[...] = xpad_ref[...] * ro[:, :, None]


def _prep_call(xpad, cnto, cnti):
    return pl.pallas_call(
        _prep_body,
        out_shape=[jax.ShapeDtypeStruct((G, NP, H), jnp.float32),
                   jax.ShapeDtypeStruct((G, NP), jnp.float32),
                   jax.ShapeDtypeStruct((G, NP), jnp.float32)],
    )(xpad, cnto, cnti)


def _layer_body(agg_ref, ri_ref, ro_ref, W_ref, b_ref, out_ref, *, do_relu):
    v = agg_ref[...] * ri_ref[...][:, :, None]
    z = v.reshape(G * NP, H) @ W_ref[...] + b_ref[...]
    if do_relu:
        z = jnp.maximum(z, 0.0)
    out_ref[...] = z.reshape(G, NP, H) * ro_ref[...][:, :, None]


def _layer_call(agg, ri, ro, W, b, do_relu):
    return pl.pallas_call(
        functools.partial(_layer_body, do_relu=do_relu),
        out_shape=jax.ShapeDtypeStruct((G, NP, H), jnp.float32),
    )(agg, ri, ro, W, b)


def _last_body(agg_ref, ri_ref, W_ref, b_ref, Wq_ref, bq_ref, Wk_ref, bk_ref,
               h_ref, q_ref, k_ref):
    v = agg_ref[...] * ri_ref[...][:, :, None]
    z = jnp.maximum(v.reshape(G * NP, H) @ W_ref[...] + b_ref[...], 0.0)
    h_ref[...] = z.reshape(G, NP, H)
    q_ref[...] = (z @ Wq_ref[...] + bq_ref[...]).reshape(G, NP, H)
    k_ref[...] = (z @ Wk_ref[...] + bk_ref[...]).reshape(G, NP, H)


def _last_call(agg, ri, W, b, Wq, bq, Wk, bk):
    return pl.pallas_call(
        _last_body,
        out_shape=[jax.ShapeDtypeStruct((G, NP, H), jnp.float32)] * 3,
    )(agg, ri, W, b, Wq, bq, Wk, bk)


_BR = 256
_NB = NP // _BR


def _att_body(q_ref, k_ref, hfull_ref, hblk_ref, Wv_ref, bv_ref, out_ref):
    br = pl.program_id(1)
    v = hfull_ref[0] @ Wv_ref[...] + bv_ref[0, 0]
    s = q_ref[0] @ k_ref[0].T
    jmask = lax.broadcasted_iota(jnp.int32, (1, NP), 1) < N
    s = jnp.where(jmask, s, -1e30)
    m = jnp.max(s, axis=1, keepdims=True)
    p = jnp.exp(s - m)
    denom = jnp.sum(p, axis=1, keepdims=True)
    fw = (p @ v) / denom
    imask = (lax.broadcasted_iota(jnp.int32, (_BR, 1), 0) + br * _BR) < N
    fw = jnp.where(imask, fw, 0.0)
    contrib = jnp.sum(hblk_ref[0] * fw, axis=0)

    @pl.when(br == 0)
    def _():
        out_ref[...] = jnp.zeros((1, 8, H), jnp.float32)
    out_ref[0, 0, :] += contrib


def _att_call(q, k, h, Wv, bv):
    return pl.pallas_call(
        _att_body,
        grid=(G, _NB),
        in_specs=[
            pl.BlockSpec((1, _BR, H), lambda g, b: (g, b, 0)),
            pl.BlockSpec((1, NP, H), lambda g, b: (g, 0, 0)),
            pl.BlockSpec((1, NP, H), lambda g, b: (g, 0, 0)),
            pl.BlockSpec((1, _BR, H), lambda g, b: (g, b, 0)),
            pl.BlockSpec((H, 1), lambda g, b: (0, 0)),
            pl.BlockSpec((1, 1), lambda g, b: (0, 0)),
        ],
        out_specs=pl.BlockSpec((1, 8, H), lambda g, b: (g, 0, 0)),
        out_shape=jax.ShapeDtypeStruct((G, 8, H), jnp.float32),
    )(q, k, h, h, Wv, bv)[:, 0, :]


def _mlp_body(z_ref, W1_ref, b1_ref, g1_ref, beta1_ref, W2_ref, b2_ref,
              g2_ref, beta2_ref, W3_ref, b3_ref, out_ref):
    def ln_leaky(z, g, b):
        mu = jnp.mean(z, axis=-1, keepdims=True)
        var = jnp.mean((z - mu) ** 2, axis=-1, keepdims=True)
        z = (z - mu) * lax.rsqrt(var + 1e-5) * g + b
        return jnp.where(z > 0, z, 0.01 * z)

    z1 = ln_leaky(z_ref[...] @ W1_ref[...] + b1_ref[...], g1_ref[...], beta1_ref[...])
    z2 = ln_leaky(z1 @ W2_ref[...] + b2_ref[...], g2_ref[...], beta2_ref[...])
    out_ref[...] = z2 @ W3_ref[...] + b3_ref[...]


def _mlp_call(z, W1, b1, g1, beta1, W2, b2, g2, beta2, W3, b3):
    return pl.pallas_call(
        _mlp_body,
        out_shape=jax.ShapeDtypeStruct((G, 1), jnp.float32),
    )(z, W1, b1, g1, beta1, W2, b2, g2, beta2, W3, b3)


def kernel(x, edge_index, add_features, W0, b0, W_gcr, b_gcr, Wq, bq, Wk, bk,
           Wv, bv, W1, b1, g1, beta1, W2, b2, g2, beta2, W3, b3):
    src = edge_index[:, 0, :].astype(jnp.int32).reshape(G, 16, EPT)
    dst = edge_index[:, 1, :].astype(jnp.int32).reshape(G, 16, EPT)

    padv = (N + (jnp.arange(EPTP - EPT, dtype=jnp.int32) % 16))
    padv = jnp.broadcast_to(padv, (G, 16, EPTP - EPT))
    srcp = jnp.concatenate([src, padv], axis=2)
    dstp = jnp.concatenate([dst, padv], axis=2)

    gidx = jnp.arange(G, dtype=jnp.int32)[:, None, None]
    slot = (gidx % 2)
    srcC = srcp + gidx * NP
    srcL = srcp + slot * NP
    dstL = dstp + slot * NP

    def regroup(a):
        return (a.reshape(2, 2, 16, EPTP).transpose(0, 2, 1, 3)
                 .reshape(2, 16, NCH, IC))
    srcC4, srcL4, dstL4 = regroup(srcC), regroup(srcL), regroup(dstL)

    dego2, degi2 = _deg_call(srcL4, dstL4)
    cnto = dego2.reshape(G, NP)
    cnti = degi2.reshape(G, NP)

    xpad = jnp.pad(x, ((0, 0), (0, NP - N), (0, 0)))
    table0, ro, ri = _prep_call(xpad, cnto, cnti)

    Ws = [W0] + [W_gcr[i] for i in range(10)]
    bs = [jnp.broadcast_to(b0, (1, H))] + \
         [jnp.broadcast_to(b_gcr[i], (1, H)) for i in range(10)]

    table = table0.reshape(G * NP, H)
    for l in range(10):
        agg = _msg_call(table, srcC4, dstL4).reshape(G, NP, H)
        table = _layer_call(agg, ri, ro, Ws[l], bs[l],
                            do_relu=(l != 0)).reshape(G * NP, H)
    agg = _msg_call(table, srcC4, dstL4).reshape(G, NP, H)
    h, q, k = _last_call(agg, ri, Ws[10], bs[10],
                         Wq, jnp.broadcast_to(bq, (1, H)),
                         Wk, jnp.broadcast_to(bk, (1, H)))

    feats = _att_call(q, k, h, Wv, jnp.reshape(bv, (1, 1)))
    z = jnp.concatenate([feats, add_features], axis=1)
    out2 = _mlp_call(z, W1, jnp.broadcast_to(b1, (1, 1024)), g1, beta1,
                     W2, jnp.broadcast_to(b2, (1, 512)), g2, beta2,
                     W3, jnp.reshape(b3, (1, 1)))
    return out2[:, 0]


# speedup vs baseline: 1.0666x; 1.0081x over previous
import functools

import jax
import jax.numpy as jnp
from jax import lax
from jax.experimental import pallas as pl
from jax.experimental.pallas import tpu as pltpu, tpu_sc as plsc

G, N, E = 4, 2500, 40000
H = 128
NP = 2560
EPT = E // 16
EPTP = NP
IC = 64
NCH = 2 * EPTP // IC
ROWS_PER_TILE = 2 * NP // 16

_mesh = plsc.VectorSubcoreMesh(core_axis_name="c", subcore_axis_name="s")


def _deg_body(srcL_hbm, dstL_hbm, dego_hbm, degi_hbm,
              sidx, didx, ones_v, zb, stage, sco, sci, osems, isems):
    c = lax.axis_index("c")
    t = lax.axis_index("s")

    for j in range(8):
        ones_v[pl.ds(j * 16, 16)] = jnp.ones((16,), jnp.float32)
    def zb16(i, _):
        zb[pl.ds(i * 16, 16)] = jnp.zeros((16,), jnp.float32)
        return 0
    lax.fori_loop(0, ROWS_PER_TILE // 16, zb16, 0)
    pltpu.sync_copy(zb, sco.at[pl.ds(t * ROWS_PER_TILE, ROWS_PER_TILE)])
    pltpu.sync_copy(zb, sci.at[pl.ds(t * ROWS_PER_TILE, ROWS_PER_TILE)])

    pltpu.sync_copy(srcL_hbm.at[c, t], sidx)
    pltpu.sync_copy(dstL_hbm.at[c, t], didx)
    plsc.subcore_barrier()

    ones16 = ones_v.at[pl.ds(0, IC)]

    def count(i, _):
        @pl.when(i >= 4)
        def _():
            pltpu.make_async_copy(ones16, sco.at[sidx.at[i - 4]],
                                  osems.at[(i - 4) % 4]).wait()
            pltpu.make_async_copy(ones16, sci.at[didx.at[i - 4]],
                                  isems.at[(i - 4) % 4]).wait()
        pltpu.async_copy(ones16, sco.at[sidx.at[i]], osems.at[i % 4],
                         add=True)
        pltpu.async_copy(ones16, sci.at[didx.at[i]], isems.at[i % 4],
                         add=True)
        return 0
    lax.fori_loop(0, NCH, count, 0)

    def drain(i, _):
        pltpu.make_async_copy(ones16, sco.at[sidx.at[i]],
                              osems.at[i % 4]).wait()
        pltpu.make_async_copy(ones16, sci.at[didx.at[i]],
                              isems.at[i % 4]).wait()
        return 0
    lax.fori_loop(NCH - 4, NCH, drain, 0)
    plsc.subcore_barrier()

    pltpu.sync_copy(sco.at[pl.ds(t * ROWS_PER_TILE, ROWS_PER_TILE)], stage)
    pltpu.sync_copy(stage, dego_hbm.at[c, t])
    pltpu.sync_copy(sci.at[pl.ds(t * ROWS_PER_TILE, ROWS_PER_TILE)], stage)
    pltpu.sync_copy(stage, degi_hbm.at[c, t])


_deg_call = pl.kernel(
    _deg_body,
    out_type=[jax.ShapeDtypeStruct((2, 16, ROWS_PER_TILE), jnp.float32),
              jax.ShapeDtypeStruct((2, 16, ROWS_PER_TILE), jnp.float32)],
    mesh=_mesh,
    scratch_types=[
        pltpu.VMEM((NCH, IC), jnp.int32),
        pltpu.VMEM((NCH, IC), jnp.int32),
        pltpu.VMEM((128,), jnp.float32),
        pltpu.VMEM((ROWS_PER_TILE,), jnp.float32),
        pltpu.VMEM((ROWS_PER_TILE,), jnp.float32),
        pltpu.VMEM_SHARED((2 * NP,), jnp.float32),
        pltpu.VMEM_SHARED((2 * NP,), jnp.float32),
        pltpu.SemaphoreType.DMA((4,)),
        pltpu.SemaphoreType.DMA((4,)),
    ],
)


NC2 = EPTP // IC
RPT2 = NP // 16


def _msg_body(table_hbm, src_hbm, dst_hbm, out_hbm,
              sidx, didx, bufs, zb, acc, gsems, ssems):
    c = lax.axis_index("c")
    t = lax.axis_index("s")

    pltpu.async_copy(src_hbm.at[c, t], sidx, gsems.at[0])
    pltpu.async_copy(dst_hbm.at[c, t], didx, gsems.at[1])

    def zfill(i, _):
        zb[i // 8, pl.ds((i % 8) * 16, 16)] = jnp.zeros((16,), jnp.float32)
        return 0
    lax.fori_loop(0, 32 * 8, zfill, 0)
    for f in range(5):
        pltpu.sync_copy(zb, acc.at[pl.ds(t * RPT2 + f * 32, 32)])
    pltpu.make_async_copy(src_hbm.at[c, t], sidx, gsems.at[0]).wait()
    pltpu.make_async_copy(dst_hbm.at[c, t], didx, gsems.at[1]).wait()
    plsc.subcore_barrier()

    def gath(i, b):
        return pltpu.async_copy(table_hbm.at[sidx.at[i]], bufs.at[b],
                                gsems.at[b])

    def scat(i, b):
        return pltpu.async_copy(bufs.at[b], acc.at[didx.at[i]],
                                ssems.at[b], add=True)

    for i in range(7):
        gath(i, i)

    def group8(j, _):
        for b in range(8):
            i = j * 8 + b
            pltpu.make_async_copy(table_hbm.at[sidx.at[i]], bufs.at[b],
                                  gsems.at[b]).wait()
            nb = (b + 7) % 8
            @pl.when(i > 0)
            def _():
                pltpu.make_async_copy(bufs.at[nb], acc.at[didx.at[i - 1]],
                                      ssems.at[nb]).wait()
            @pl.when(i + 7 < NC2)
            def _():
                gath(i + 7, nb)
            scat(i, b)
        return 0
    lax.fori_loop(0, NC2 // 8, group8, 0)
    pltpu.make_async_copy(bufs.at[7], acc.at[didx.at[NC2 - 1]],
                          ssems.at[7]).wait()
    plsc.subcore_barrier()

    for f in range(2):
        base = t * RPT2 + f * 64
        pltpu.sync_copy(acc.at[pl.ds(base, 64)], bufs.at[f])
        pltpu.async_copy(bufs.at[f], out_hbm.at[c, pl.ds(base, 64)],
                         gsems.at[f])
    base = t * RPT2 + 128
    pltpu.sync_copy(acc.at[pl.ds(base, 32)], bufs.at[2].at[pl.ds(0, 32)])
    pltpu.sync_copy(bufs.at[2].at[pl.ds(0, 32)],
                    out_hbm.at[c, pl.ds(base, 32)])
    for f in range(2):
        pltpu.make_async_copy(bufs.at[f],
                              out_hbm.at[c, pl.ds(t * RPT2 + f * 64, 64)],
                              gsems.at[f]).wait()


_msg_call = pl.kernel(
    _msg_body,
    out_type=jax.ShapeDtypeStruct((2, NP, H), jnp.float32),
    mesh=_mesh,
    scratch_types=[
        pltpu.VMEM((NC2, IC), jnp.int32),
        pltpu.VMEM((NC2, IC), jnp.int32),
        pltpu.VMEM((8, IC, H), jnp.float32),
        pltpu.VMEM((32, H), jnp.float32),
        pltpu.VMEM_SHARED((NP, H), jnp.float32),
        pltpu.SemaphoreType.DMA((8,)),
        pltpu.SemaphoreType.DMA((8,)),
    ],
)


GNP = G * NP


def _prep_body(xpad_ref, cnto_ref, cnti_ref, t0_ref, ro_ref, ri_ref):
    ro = lax.rsqrt(jnp.maximum(cnto_ref[...], 1.0))
    ri = lax.rsqrt(jnp.maximum(cnti_ref[...], 1.0))
    ro_ref[...] = ro
    ri_ref[...] = ri
    t0_ref[...] = xpad_ref[...] * ro[:, :, None]


def _prep_call(xpad, cnto, cnti):
    return pl.pallas_call(
        _prep_body,
        out_shape=[jax.ShapeDtypeStruct((G, NP, H), jnp.float32),
                   jax.ShapeDtypeStruct((G, NP), jnp.float32),
                   jax.ShapeDtypeStruct((G, NP), jnp.float32)],
    )(xpad, cnto, cnti)


def _layer_body(agg_ref, ri_ref, ro_ref, W_ref, b_ref, out_ref, *, do_relu):
    v = agg_ref[...] * ri_ref[...][:, :, None]
    z = v.reshape(2 * NP, H) @ W_ref[...] + b_ref[...]
    if do_relu:
        z = jnp.maximum(z, 0.0)
    out_ref[...] = z.reshape(2, NP, H) * ro_ref[...][:, :, None]


def _layer_call(agg, ri, ro, W, b, do_relu):
    return pl.pallas_call(
        functools.partial(_layer_body, do_relu=do_relu),
        out_shape=jax.ShapeDtypeStruct((2, NP, H), jnp.float32),
    )(agg, ri, ro, W, b)


def _last_body(agg_ref, ri_ref, W_ref, b_ref, Wq_ref, bq_ref, Wk_ref, bk_ref,
               h_ref, q_ref, k_ref):
    v = agg_ref[...] * ri_ref[...][:, :, None]
    z = jnp.maximum(v.reshape(2 * NP, H) @ W_ref[...] + b_ref[...], 0.0)
    h_ref[...] = z.reshape(2, NP, H)
    q_ref[...] = (z @ Wq_ref[...] + bq_ref[...]).reshape(2, NP, H)
    k_ref[...] = (z @ Wk_ref[...] + bk_ref[...]).reshape(2, NP, H)


def _last_call(agg, ri, W, b, Wq, bq, Wk, bk):
    return pl.pallas_call(
        _last_body,
        out_shape=[jax.ShapeDtypeStruct((2, NP, H), jnp.float32)] * 3,
    )(agg, ri, W, b, Wq, bq, Wk, bk)


_BR = 256
_NB = NP // _BR


def _att_body(q_ref, k_ref, hfull_ref, hblk_ref, Wv_ref, bv_ref, out_ref):
    br = pl.program_id(1)
    v = hfull_ref[0] @ Wv_ref[...] + bv_ref[0, 0]
    s = q_ref[0] @ k_ref[0].T
    jmask = lax.broadcasted_iota(jnp.int32, (1, NP), 1) < N
    s = jnp.where(jmask, s, -1e30)
    m = jnp.max(s, axis=1, keepdims=True)
    p = jnp.exp(s - m)
    denom = jnp.sum(p, axis=1, keepdims=True)
    fw = (p @ v) / denom
    imask = (lax.broadcasted_iota(jnp.int32, (_BR, 1), 0) + br * _BR) < N
    fw = jnp.where(imask, fw, 0.0)
    contrib = jnp.sum(hblk_ref[0] * fw, axis=0)

    @pl.when(br == 0)
    def _():
        out_ref[...] = jnp.zeros((1, 8, H), jnp.float32)
    out_ref[0, 0, :] += contrib


def _att_call(q, k, h, Wv, bv):
    return pl.pallas_call(
        _att_body,
        grid=(2, _NB),
        in_specs=[
            pl.BlockSpec((1, _BR, H), lambda g, b: (g, b, 0)),
            pl.BlockSpec((1, NP, H), lambda g, b: (g, 0, 0)),
            pl.BlockSpec((1, NP, H), lambda g, b: (g, 0, 0)),
            pl.BlockSpec((1, _BR, H), lambda g, b: (g, b, 0)),
            pl.BlockSpec((H, 1), lambda g, b: (0, 0)),
            pl.BlockSpec((1, 1), lambda g, b: (0, 0)),
        ],
        out_specs=pl.BlockSpec((1, 8, H), lambda g, b: (g, 0, 0)),
        out_shape=jax.ShapeDtypeStruct((2, 8, H), jnp.float32),
    )(q, k, h, h, Wv, bv)[:, 0, :]


def _mlp_body(z_ref, W1_ref, b1_ref, g1_ref, beta1_ref, W2_ref, b2_ref,
              g2_ref, beta2_ref, W3_ref, b3_ref, out_ref):
    def ln_leaky(z, g, b):
        mu = jnp.mean(z, axis=-1, keepdims=True)
        var = jnp.mean((z - mu) ** 2, axis=-1, keepdims=True)
        z = (z - mu) * lax.rsqrt(var + 1e-5) * g + b
        return jnp.where(z > 0, z, 0.01 * z)

    z1 = ln_leaky(z_ref[...] @ W1_ref[...] + b1_ref[...], g1_ref[...], beta1_ref[...])
    z2 = ln_leaky(z1 @ W2_ref[...] + b2_ref[...], g2_ref[...], beta2_ref[...])
    out_ref[...] = z2 @ W3_ref[...] + b3_ref[...]


def _mlp_call(z, W1, b1, g1, beta1, W2, b2, g2, beta2, W3, b3):
    return pl.pallas_call(
        _mlp_body,
        out_shape=jax.ShapeDtypeStruct((G, 1), jnp.float32),
    )(z, W1, b1, g1, beta1, W2, b2, g2, beta2, W3, b3)


def kernel(x, edge_index, add_features, W0, b0, W_gcr, b_gcr, Wq, bq, Wk, bk,
           Wv, bv, W1, b1, g1, beta1, W2, b2, g2, beta2, W3, b3):
    src = edge_index[:, 0, :].astype(jnp.int32).reshape(G, 16, EPT)
    dst = edge_index[:, 1, :].astype(jnp.int32).reshape(G, 16, EPT)

    padv = (N + (jnp.arange(EPTP - EPT, dtype=jnp.int32) % 16))
    padv = jnp.broadcast_to(padv, (G, 16, EPTP - EPT))
    srcp = jnp.concatenate([src, padv], axis=2)
    dstp = jnp.concatenate([dst, padv], axis=2)

    gidx = jnp.arange(G, dtype=jnp.int32)[:, None, None]
    slot = (gidx % 2)
    srcL = srcp + slot * NP
    dstL = dstp + slot * NP

    def regroup(a):
        return (a.reshape(2, 2, 16, EPTP).transpose(0, 2, 1, 3)
                 .reshape(2, 16, NCH, IC))
    srcL4, dstL4 = regroup(srcL), regroup(dstL)

    srcP = srcL.reshape(2, 2, 16, NC2, IC)
    dstP = dstp.reshape(2, 2, 16, NC2, IC)

    dego2, degi2 = _deg_call(srcL4, dstL4)
    cnto = dego2.reshape(G, NP)
    cnti = degi2.reshape(G, NP)

    xpad = jnp.pad(x, ((0, 0), (0, NP - N), (0, 0)))
    table0, ro, ri = _prep_call(xpad, cnto, cnti)

    Ws = [W0] + [W_gcr[i] for i in range(10)]
    bs = [jnp.broadcast_to(b0, (1, H))] + \
         [jnp.broadcast_to(b_gcr[i], (1, H)) for i in range(10)]

    tabs = [table0[0:2].reshape(2 * NP, H), table0[2:4].reshape(2 * NP, H)]
    ris = [ri[0:2], ri[2:4]]
    ros = [ro[0:2], ro[2:4]]
    bq2 = jnp.broadcast_to(bq, (1, H))
    bk2 = jnp.broadcast_to(bk, (1, H))
    bv2 = jnp.reshape(bv, (1, 1))

    for l in range(10):
        for p in range(2):
            agg = _msg_call(tabs[p], srcP[p], dstP[p]).reshape(2, NP, H)
            tabs[p] = _layer_call(agg, ris[p], ros[p], Ws[l], bs[l],
                                  do_relu=(l != 0)).reshape(2 * NP, H)
    fps = []
    for p in range(2):
        agg = _msg_call(tabs[p], srcP[p], dstP[p]).reshape(2, NP, H)
        h, q, k = _last_call(agg, ris[p], Ws[10], bs[10], Wq, bq2, Wk, bk2)
        fps.append(_att_call(q, k, h, Wv, bv2))
    feats = jnp.concatenate(fps, axis=0)
    z = jnp.concatenate([feats, add_features], axis=1)
    out2 = _mlp_call(z, W1, jnp.broadcast_to(b1, (1, 1024)), g1, beta1,
                     W2, jnp.broadcast_to(b2, (1, 512)), g2, beta2,
                     W3, jnp.reshape(b3, (1, 1)))
    return out2[:, 0]
